# Initial kernel scaffold; baseline (speedup 1.0000x reference)
import jax
import jax.numpy as jnp
from jax.experimental import pallas as pl


def kernel(q, k, v, wk, bk, wv, bv, wg, bg):
    raise NotImplementedError("write your pallas kernel here")


# speedup vs baseline: 2.0090x; 2.0090x over previous
import numpy as np
import jax
import jax.numpy as jnp
from jax.experimental import pallas as pl
from jax.experimental.pallas import tpu as pltpu

B, S, H, DIM = 1, 2048, 12, 64
D_STRIDE, L_CMP, L_SLC = 64, 64, 128
WINDOW, BLOCK_Q, TOP_K = 128, 64, 8
N_CMP = (S - L_CMP) // D_STRIDE + 1
N_SLC = (S - L_SLC) // D_STRIDE + 1
QSTEP = 256
NQS = S // QSTEP
NSEL = QSTEP // BLOCK_Q
WGRP = 128
NWIN = QSTEP // WGRP
WIN_SLAB = WGRP + 2 * WINDOW
NEG = float(np.finfo(np.float32).min)
SCALE = DIM ** (-0.5)


def _slc_map_np():
    alpha, beta = L_SLC // D_STRIDE, L_CMP // D_STRIDE
    M = np.zeros((N_CMP, N_SLC), np.float32)
    for j in range(N_SLC):
        for m in range(alpha):
            for nn in range(beta):
                idx = alpha * j - m - nn
                if 0 <= idx < N_CMP:
                    M[idx, j] += 1.0
    return M


def _softmax(x):
    m = jnp.max(x, axis=-1, keepdims=True)
    e = jnp.exp(x - m)
    return e / jnp.sum(e, axis=-1, keepdims=True)


_DN_NT = (((1,), (1,)), ((), ()))
_DN_NN = (((1,), (0,)), ((), ()))


def _dot_nt(a, b):
    return jax.lax.dot_general(a.astype(jnp.bfloat16), b.astype(jnp.bfloat16),
                               _DN_NT, preferred_element_type=jnp.float32)


def _nsa_body(q_ref, k_ref, v_ref, wkt_ref, wvt_ref, bkv_ref, wg_ref, bg_ref,
              m_ref, o_ref, kc_ref, vc_ref):
    qb = pl.program_id(1)

    @pl.when(qb == 0)
    def _compress():
        row = jax.lax.broadcasted_iota(jnp.int32, (N_CMP, S), 0)
        col = jax.lax.broadcasted_iota(jnp.int32, (N_CMP, S), 1)
        sel = (col // D_STRIDE) == row
        wkm = jnp.where(sel, wkt_ref[...], 0.0).astype(jnp.bfloat16)
        wvm = jnp.where(sel, wvt_ref[...], 0.0).astype(jnp.bfloat16)
        kc_ref[...] = (jax.lax.dot_general(wkm, k_ref[0].astype(jnp.bfloat16),
                                           _DN_NN,
                                           preferred_element_type=jnp.float32)
                       + bkv_ref[0, 0])
        vc_ref[...] = (jax.lax.dot_general(wvm, v_ref[0].astype(jnp.bfloat16),
                                           _DN_NN,
                                           preferred_element_type=jnp.float32)
                       + bkv_ref[0, 1])

    q_blk = q_ref[0]
    g0 = qb * QSTEP

    lc = _dot_nt(q_blk, kc_ref[...]) / SCALE
    lws, vwins = [], []
    for w in range(NWIN):
        wstart = g0 + w * WGRP
        kstart = jnp.clip(wstart - WINDOW, 0, S - WIN_SLAB)
        kwin = k_ref[0, pl.ds(kstart, WIN_SLAB), :]
        vwins.append(v_ref[0, pl.ds(kstart, WIN_SLAB), :])
        qw = q_blk[w * WGRP:(w + 1) * WGRP, :]
        lw = _dot_nt(qw, kwin) * SCALE
        gi = wstart + jax.lax.broadcasted_iota(jnp.int32, (WGRP, WIN_SLAB), 0)
        kj = kstart + jax.lax.broadcasted_iota(jnp.int32, (WGRP, WIN_SLAB), 1)
        lws.append(jnp.where((kj >= gi - WINDOW) & (kj <= gi + WINDOW),
                             lw, NEG))
    gl = _dot_nt(q_blk, wg_ref[...]) + bg_ref[...]

    pc = _softmax(lc)
    pcb = pc.astype(jnp.bfloat16)
    out_cmp = jax.lax.dot_general(pcb, vc_ref[...].astype(jnp.bfloat16),
                                  _DN_NN, preferred_element_type=jnp.float32)

    out_win = jnp.concatenate(
        [jax.lax.dot_general(_softmax(lws[w]).astype(jnp.bfloat16),
                             vwins[w].astype(jnp.bfloat16), _DN_NN,
                             preferred_element_type=jnp.float32)
         for w in range(NWIN)], axis=0)

    pslc = jax.lax.dot_general(pcb, m_ref[...], _DN_NN,
                               preferred_element_type=jnp.float32)
    iota_l = jax.lax.broadcasted_iota(jnp.int32, (1, N_SLC), 1)
    rr = jax.lax.broadcasted_iota(jnp.int32, (N_SLC, N_SLC), 0)
    cc = jax.lax.broadcasted_iota(jnp.int32, (N_SLC, N_SLC), 1)
    out_slc_parts = []
    for u in range(NSEL):
        scores = jnp.sum(pslc[u * BLOCK_Q:(u + 1) * BLOCK_Q, :],
                         axis=0, keepdims=True)
        st = jnp.transpose(scores)
        above = (st > scores) | ((st == scores) & (rr < cc))
        rank = jnp.sum(above.astype(jnp.float32), axis=0, keepdims=True)
        qh = q_blk[u * BLOCK_Q:(u + 1) * BLOCK_Q, :]
        js, lgs = [], []
        for i in range(TOP_K):
            j = jnp.min(jnp.where(rank == float(i), iota_l, N_SLC))
            js.append(j)
            lgs.append(_dot_nt(qh, k_ref[0, pl.ds(j * D_STRIDE, L_SLC), :]))
        ls = jnp.concatenate(lgs, axis=1) * SCALE
        psb = _softmax(ls).astype(jnp.bfloat16)
        acc = jnp.zeros((BLOCK_Q, DIM), jnp.float32)
        for i in range(TOP_K):
            vblk = v_ref[0, pl.ds(js[i] * D_STRIDE, L_SLC), :]
            acc = acc + jax.lax.dot_general(
                psb[:, i * L_SLC:(i + 1) * L_SLC], vblk.astype(jnp.bfloat16),
                _DN_NN, preferred_element_type=jnp.float32)
        out_slc_parts.append(acc)
    out_slc = jnp.concatenate(out_slc_parts, axis=0)

    g = jax.nn.sigmoid(gl)
    o_ref[0] = (g[:, 0:1] * out_cmp + g[:, 1:2] * out_slc
                + g[:, 2:3] * out_win)


def kernel(q, k, v, wk, bk, wv, bv, wg, bg):
    qT = jnp.transpose(q[0], (1, 0, 2)).astype(jnp.bfloat16)
    kT = jnp.transpose(k[0], (1, 0, 2))
    vT = jnp.transpose(v[0], (1, 0, 2))
    wkt = jnp.tile(wk, N_CMP).reshape(1, S)
    wvt = jnp.tile(wv, N_CMP).reshape(1, S)
    bkv = jnp.stack([bk, bv]).reshape(1, 2)
    bg2 = bg.reshape(1, 3)
    wgb = wg.astype(jnp.bfloat16)
    Mmap = jnp.asarray(_slc_map_np()).astype(jnp.bfloat16)

    out = pl.pallas_call(
        _nsa_body,
        grid=(H, NQS),
        in_specs=[
            pl.BlockSpec((1, QSTEP, DIM), lambda h, qb: (h, qb, 0)),
            pl.BlockSpec((1, S, DIM), lambda h, qb: (h, 0, 0)),
            pl.BlockSpec((1, S, DIM), lambda h, qb: (h, 0, 0)),
            pl.BlockSpec((1, S), lambda h, qb: (0, 0)),
            pl.BlockSpec((1, S), lambda h, qb: (0, 0)),
            pl.BlockSpec((1, 2), lambda h, qb: (0, 0)),
            pl.BlockSpec((3, DIM), lambda h, qb: (0, 0)),
            pl.BlockSpec((1, 3), lambda h, qb: (0, 0)),
            pl.BlockSpec((N_CMP, N_SLC), lambda h, qb: (0, 0)),
        ],
        out_specs=pl.BlockSpec((1, QSTEP, DIM), lambda h, qb: (h, qb, 0)),
        out_shape=jax.ShapeDtypeStruct((H, S, DIM), jnp.float32),
        scratch_shapes=[
            pltpu.VMEM((N_CMP, DIM), jnp.float32),
            pltpu.VMEM((N_CMP, DIM), jnp.float32),
        ],
    )(qT, kT, vT, wkt, wvt, bkv, wgb, bg2, Mmap)
    return jnp.transpose(out, (1, 0, 2))[None]


# speedup vs baseline: 4.0231x; 2.0025x over previous
import numpy as np
import jax
import jax.numpy as jnp
from jax.experimental import pallas as pl
from jax.experimental.pallas import tpu as pltpu

B, S, H, DIM = 1, 2048, 12, 64
D_STRIDE, L_CMP, L_SLC = 64, 64, 128
WINDOW, BLOCK_Q, TOP_K = 128, 64, 8
N_CMP = (S - L_CMP) // D_STRIDE + 1
N_SLC = (S - L_SLC) // D_STRIDE + 1
QSTEP = 1024
NQS = S // QSTEP
NSEL = QSTEP // BLOCK_Q
WGRP = 128
NWIN = QSTEP // WGRP
WIN_SLAB = WGRP + 2 * WINDOW
NEG = float(np.finfo(np.float32).min)
SCALE = DIM ** (-0.5)


def _slc_map_np():
    alpha, beta = L_SLC // D_STRIDE, L_CMP // D_STRIDE
    M = np.zeros((N_CMP, N_SLC), np.float32)
    for j in range(N_SLC):
        for m in range(alpha):
            for nn in range(beta):
                idx = alpha * j - m - nn
                if 0 <= idx < N_CMP:
                    M[idx, j] += 1.0
    return M


def _softmax(x):
    m = jnp.max(x, axis=-1, keepdims=True)
    e = jnp.exp(x - m)
    return e / jnp.sum(e, axis=-1, keepdims=True)


_DN_NT = (((1,), (1,)), ((), ()))
_DN_NN = (((1,), (0,)), ((), ()))


def _dot_nt(a, b):
    return jax.lax.dot_general(a.astype(jnp.bfloat16), b.astype(jnp.bfloat16),
                               _DN_NT, preferred_element_type=jnp.float32)


def _nsa_body(q_ref, k_ref, v_ref, wkt_ref, wvt_ref, bkv_ref, wg_ref, bg_ref,
              m_ref, o_ref, kc_ref, vc_ref):
    qb = pl.program_id(1)

    @pl.when(qb == 0)
    def _compress():
        row = jax.lax.broadcasted_iota(jnp.int32, (N_CMP, S), 0)
        col = jax.lax.broadcasted_iota(jnp.int32, (N_CMP, S), 1)
        sel = (col // D_STRIDE) == row
        wkm = jnp.where(sel, wkt_ref[...], 0.0).astype(jnp.bfloat16)
        wvm = jnp.where(sel, wvt_ref[...], 0.0).astype(jnp.bfloat16)
        kc_ref[...] = (jax.lax.dot_general(wkm, k_ref[0].astype(jnp.bfloat16),
                                           _DN_NN,
                                           preferred_element_type=jnp.float32)
                       + bkv_ref[0, 0])
        vc_ref[...] = (jax.lax.dot_general(wvm, v_ref[0].astype(jnp.bfloat16),
                                           _DN_NN,
                                           preferred_element_type=jnp.float32)
                       + bkv_ref[0, 1])

    q_blk = q_ref[0]
    g0 = qb * QSTEP

    kwins, vwins, kstarts = [], [], []
    for w in range(NWIN):
        wstart = g0 + w * WGRP
        kstart = jnp.clip(wstart - WINDOW, 0, S - WIN_SLAB)
        kstarts.append(kstart)
        kwins.append(k_ref[0, pl.ds(kstart, WIN_SLAB), :].astype(jnp.bfloat16))
        vwins.append(v_ref[0, pl.ds(kstart, WIN_SLAB), :].astype(jnp.bfloat16))

    lc = _dot_nt(q_blk, kc_ref[...]) / SCALE
    ee = (jax.lax.broadcasted_iota(jnp.int32, (WGRP, WIN_SLAB), 1)
          - jax.lax.broadcasted_iota(jnp.int32, (WGRP, WIN_SLAB), 0))
    lws = []
    for w in range(NWIN):
        off = (g0 + w * WGRP) - kstarts[w]
        qw = q_blk[w * WGRP:(w + 1) * WGRP, :]
        lw = _dot_nt(qw, kwins[w]) * SCALE
        lws.append(jnp.where((ee >= off - WINDOW) & (ee <= off + WINDOW),
                             lw, NEG))
    gl = _dot_nt(q_blk, wg_ref[...]) + bg_ref[...]

    pc = _softmax(lc)
    pcb = pc.astype(jnp.bfloat16)
    pslc = jax.lax.dot_general(pcb, m_ref[...], _DN_NN,
                               preferred_element_type=jnp.float32)
    iota_l = jax.lax.broadcasted_iota(jnp.int32, (1, N_SLC), 1)
    rr = jax.lax.broadcasted_iota(jnp.int32, (N_SLC, N_SLC), 0)
    cc = jax.lax.broadcasted_iota(jnp.int32, (N_SLC, N_SLC), 1)
    js_all = []
    for u in range(NSEL):
        scores = jnp.sum(pslc[u * BLOCK_Q:(u + 1) * BLOCK_Q, :],
                         axis=0, keepdims=True)
        st = jnp.transpose(scores)
        above = (st > scores) | ((st == scores) & (rr < cc))
        rank = jnp.sum(above.astype(jnp.float32), axis=0, keepdims=True)
        js_all.append([jnp.min(jnp.where(rank == float(i), iota_l, N_SLC))
                       for i in range(TOP_K)])

    out_cmp = jax.lax.dot_general(pcb, vc_ref[...].astype(jnp.bfloat16),
                                  _DN_NN, preferred_element_type=jnp.float32)

    out_win = jnp.concatenate(
        [jax.lax.dot_general(_softmax(lws[w]).astype(jnp.bfloat16),
                             vwins[w].astype(jnp.bfloat16), _DN_NN,
                             preferred_element_type=jnp.float32)
         for w in range(NWIN)], axis=0)

    lgs_all = [[None] * TOP_K for _ in range(NSEL)]
    for i in range(TOP_K):
        for u in range(NSEL):
            qh = q_blk[u * BLOCK_Q:(u + 1) * BLOCK_Q, :]
            lgs_all[u][i] = _dot_nt(
                qh, k_ref[0, pl.ds(js_all[u][i] * D_STRIDE, L_SLC), :])
    out_slc_parts = []
    for u in range(NSEL):
        ls = jnp.concatenate(lgs_all[u], axis=1) * SCALE
        psb = _softmax(ls).astype(jnp.bfloat16)
        acc = jnp.zeros((BLOCK_Q, DIM), jnp.float32)
        for i in range(TOP_K):
            vblk = v_ref[0, pl.ds(js_all[u][i] * D_STRIDE, L_SLC), :]
            acc = acc + jax.lax.dot_general(
                psb[:, i * L_SLC:(i + 1) * L_SLC], vblk.astype(jnp.bfloat16),
                _DN_NN, preferred_element_type=jnp.float32)
        out_slc_parts.append(acc)
    out_slc = jnp.concatenate(out_slc_parts, axis=0)

    g = jax.nn.sigmoid(gl)
    o_ref[0] = (g[:, 0:1] * out_cmp + g[:, 1:2] * out_slc
                + g[:, 2:3] * out_win)


def kernel(q, k, v, wk, bk, wv, bv, wg, bg):
    qT = jnp.transpose(q[0], (1, 0, 2)).astype(jnp.bfloat16)
    kT = jnp.transpose(k[0], (1, 0, 2))
    vT = jnp.transpose(v[0], (1, 0, 2))
    wkt = jnp.tile(wk, N_CMP).reshape(1, S)
    wvt = jnp.tile(wv, N_CMP).reshape(1, S)
    bkv = jnp.stack([bk, bv]).reshape(1, 2)
    bg2 = bg.reshape(1, 3)
    wgb = wg.astype(jnp.bfloat16)
    Mmap = jnp.asarray(_slc_map_np()).astype(jnp.bfloat16)

    out = pl.pallas_call(
        _nsa_body,
        grid=(H, NQS),
        in_specs=[
            pl.BlockSpec((1, QSTEP, DIM), lambda h, qb: (h, qb, 0)),
            pl.BlockSpec((1, S, DIM), lambda h, qb: (h, 0, 0)),
            pl.BlockSpec((1, S, DIM), lambda h, qb: (h, 0, 0)),
            pl.BlockSpec((1, S), lambda h, qb: (0, 0)),
            pl.BlockSpec((1, S), lambda h, qb: (0, 0)),
            pl.BlockSpec((1, 2), lambda h, qb: (0, 0)),
            pl.BlockSpec((3, DIM), lambda h, qb: (0, 0)),
            pl.BlockSpec((1, 3), lambda h, qb: (0, 0)),
            pl.BlockSpec((N_CMP, N_SLC), lambda h, qb: (0, 0)),
        ],
        out_specs=pl.BlockSpec((1, QSTEP, DIM), lambda h, qb: (h, qb, 0)),
        out_shape=jax.ShapeDtypeStruct((H, S, DIM), jnp.float32),
        scratch_shapes=[
            pltpu.VMEM((N_CMP, DIM), jnp.float32),
            pltpu.VMEM((N_CMP, DIM), jnp.float32),
        ],
    )(qT, kT, vT, wkt, wvt, bkv, wgb, bg2, Mmap)
    return jnp.transpose(out, (1, 0, 2))[None]
